# baseline (speedup 1.0000x reference)
import functools

import jax
import jax.numpy as jnp
from jax import lax
from jax.experimental import pallas as pl
from jax.experimental.pallas import tpu as pltpu
from jax.experimental.pallas import tpu_sc as plsc

_NUM_CLASSES = 100000
_NUM_FEATURES = 64
_BATCH = 1024
_BLK = 2000
_NBLK = _NUM_CLASSES // _BLK
_INV_TEMP = 20.0


def _gather_target_rows(bank2, idx):
    info = plsc.get_sparse_core_info()
    nw = info.num_cores * info.num_subcores
    b_per_w = _BATCH // nw
    mesh = plsc.VectorSubcoreMesh(core_axis_name="c", subcore_axis_name="s")

    @functools.partial(
        pl.kernel, mesh=mesh,
        out_type=jax.ShapeDtypeStruct((_BATCH, 2 * _NUM_FEATURES), jnp.float32),
        scratch_types=[
            pltpu.VMEM((b_per_w,), jnp.int32),
            pltpu.VMEM((b_per_w, 2 * _NUM_FEATURES), jnp.float32),
            pltpu.SemaphoreType.DMA,
        ],
    )
    def k(table_hbm, idx_hbm, out_hbm, idx_v, rows_v, sem):
        wid = lax.axis_index("s") * info.num_cores + lax.axis_index("c")
        base = wid * b_per_w
        pltpu.sync_copy(idx_hbm.at[pl.ds(base, b_per_w)], idx_v)
        pltpu.async_copy(table_hbm.at[idx_v], rows_v, sem).wait()
        pltpu.sync_copy(rows_v, out_hbm.at[pl.ds(base, b_per_w)])

    return k(bank2, idx)


def _loss_kernel(x_ref, bank_ref, rows_ref, par_ref, out_ref, ni_ref, m_ref, s_ref):
    j = pl.program_id(0)

    @pl.when(j == 0)
    def _init():
        x = x_ref[...]
        nrm = jnp.sqrt(jnp.sum(x * x, axis=1, keepdims=True))
        ni_ref[...] = (x * (_INV_TEMP / jnp.maximum(nrm, 1e-12))).astype(jnp.bfloat16)
        m_ref[...] = jnp.full((1, _BATCH), -1e30, jnp.float32)
        s_ref[...] = jnp.zeros((1, _BATCH), jnp.float32)

    bank = bank_ref[...].astype(jnp.bfloat16)
    ni = ni_ref[...]
    logits = lax.dot_general(
        bank, ni, (((1,), (1,)), ((), ())),
        preferred_element_type=jnp.float32)
    m_old = m_ref[...]
    m_new = jnp.maximum(m_old, jnp.max(logits, axis=0, keepdims=True))
    p = jnp.exp(logits - m_new)
    s_ref[...] = s_ref[...] * jnp.exp(m_old - m_new) + jnp.sum(p, axis=0, keepdims=True)
    m_ref[...] = m_new

    @pl.when(j == _NBLK - 1)
    def _fin():
        lse_sum = jnp.sum(m_ref[...] + jnp.log(s_ref[...]))
        odd = par_ref[...] != 0
        row = jnp.where(odd, rows_ref[:, _NUM_FEATURES:], rows_ref[:, :_NUM_FEATURES])
        tgt_sum = jnp.sum(row * ni_ref[...].astype(jnp.float32))
        out_ref[0, 0] = (lse_sum - tgt_sum) * (1.0 / _BATCH)


def kernel(inputs, targets, features_bank):
    tgt = targets.astype(jnp.int32)
    bank2 = features_bank.reshape(_NUM_CLASSES // 2, 2 * _NUM_FEATURES)
    rows = _gather_target_rows(bank2, tgt // 2)
    loss = pl.pallas_call(
        _loss_kernel,
        grid=(_NBLK,),
        in_specs=[
            pl.BlockSpec((_BATCH, _NUM_FEATURES), lambda j: (0, 0)),
            pl.BlockSpec((_BLK, _NUM_FEATURES), lambda j: (j, 0)),
            pl.BlockSpec((_BATCH, 2 * _NUM_FEATURES), lambda j: (0, 0)),
            pl.BlockSpec((_BATCH, 1), lambda j: (0, 0)),
        ],
        out_specs=pl.BlockSpec(memory_space=pltpu.SMEM),
        out_shape=jax.ShapeDtypeStruct((1, 1), jnp.float32),
        scratch_shapes=[
            pltpu.VMEM((_BATCH, _NUM_FEATURES), jnp.bfloat16),
            pltpu.VMEM((1, _BATCH), jnp.float32),
            pltpu.VMEM((1, _BATCH), jnp.float32),
        ],
    )(inputs, features_bank, rows, (tgt % 2).reshape(_BATCH, 1))
    return loss[0, 0]
